# Initial kernel scaffold; baseline (speedup 1.0000x reference)
import jax
import jax.numpy as jnp
from jax.experimental import pallas as pl


def kernel(x, edge_index, batch, W1a, b1a, W1b, b1b, W2a, b2a, W2b, b2b, Wc, bc):
    raise NotImplementedError("write your pallas kernel here")


# speedup vs baseline: 3.2241x; 3.2241x over previous
import functools

import jax
import jax.numpy as jnp
from jax import lax
from jax.experimental import pallas as pl
from jax.experimental.pallas import tpu as pltpu
from jax.experimental.pallas import tpu_sc as plsc

_N = 10000
_E = 320000
_D = 128
_HALF = 64
_G = 64
_C = 10
_NC = 2
_NS = 16
_K = 80
_EPT = _E // _NS
_CH = _EPT // _K
_RPT = 624
_TAIL0 = _NS * _RPT
_TAILN = _N - _TAIL0

_BN = 400
_NB = _N // _BN

_sc_mesh = plsc.VectorSubcoreMesh(core_axis_name="c", subcore_axis_name="s")


@functools.partial(
    pl.kernel,
    out_type=jax.ShapeDtypeStruct((_NC, _N, _HALF), jnp.float32),
    mesh=_sc_mesh,
    scratch_types=[
        pltpu.VMEM_SHARED((_N, _HALF), jnp.float32),
        pltpu.VMEM((_K,), jnp.int32),
        pltpu.VMEM((_K,), jnp.int32),
        pltpu.VMEM((_K, _HALF), jnp.float32),
        pltpu.SemaphoreType.DMA,
    ],
    compiler_params=pltpu.CompilerParams(use_tc_tiling_on_sc=False),
)
def _sc_agg(h2_hbm, src2_hbm, dst_hbm, z_hbm, acc, sidx, didx, rows, sem):
    c = lax.axis_index("c")
    s = lax.axis_index("s")
    r0 = s * _RPT
    pltpu.sync_copy(h2_hbm.at[pl.ds(c * _N + r0, _RPT)], acc.at[pl.ds(r0, _RPT)])

    @pl.when(s == 0)
    def _():
        pltpu.sync_copy(h2_hbm.at[pl.ds(c * _N + _TAIL0, _TAILN)],
                        acc.at[pl.ds(_TAIL0, _TAILN)])

    plsc.subcore_barrier()
    ebase = s * _EPT

    def body(j, carry):
        o = ebase + j * _K
        pltpu.sync_copy(src2_hbm.at[pl.ds(c * _E + o, _K)], sidx)
        pltpu.sync_copy(dst_hbm.at[pl.ds(o, _K)], didx)
        pltpu.async_copy(h2_hbm.at[sidx], rows, sem).wait()
        pltpu.sync_copy(rows, acc.at[didx], add=True)
        return carry

    lax.fori_loop(0, _CH, body, 0)
    plsc.subcore_barrier()
    pltpu.sync_copy(acc.at[pl.ds(r0, _RPT)], z_hbm.at[c, pl.ds(r0, _RPT)])

    @pl.when(s == 0)
    def _():
        pltpu.sync_copy(acc.at[pl.ds(_TAIL0, _TAILN)],
                        z_hbm.at[c, pl.ds(_TAIL0, _TAILN)])


def _mlp_body(z_ref, wa_ref, ba_ref, wb_ref, bb_ref, h_ref):
    z = jnp.concatenate([z_ref[0], z_ref[1]], axis=-1)
    a = jnp.maximum(
        jnp.dot(z, wa_ref[...], preferred_element_type=jnp.float32) + ba_ref[...], 0.0)
    h = jnp.maximum(
        jnp.dot(a, wb_ref[...], preferred_element_type=jnp.float32) + bb_ref[...], 0.0)
    h_ref[0] = h[:, :_HALF]
    h_ref[1] = h[:, _HALF:]


def _tc_mlp(z, Wa, ba, Wb, bb):
    return pl.pallas_call(
        _mlp_body,
        grid=(_NB,),
        in_specs=[
            pl.BlockSpec((_NC, _BN, _HALF), lambda b: (0, b, 0)),
            pl.BlockSpec((_D, _D), lambda b: (0, 0)),
            pl.BlockSpec((1, _D), lambda b: (0, 0)),
            pl.BlockSpec((_D, _D), lambda b: (0, 0)),
            pl.BlockSpec((1, _D), lambda b: (0, 0)),
        ],
        out_specs=pl.BlockSpec((_NC, _BN, _HALF), lambda b: (0, b, 0)),
        out_shape=jax.ShapeDtypeStruct((_NC, _N, _HALF), jnp.float32),
    )(z, Wa, ba.reshape(1, _D), Wb, bb.reshape(1, _D))


def _final_body(z_ref, batch_ref, wa_ref, ba_ref, wb_ref, bb_ref, wc_ref,
                bc_ref, out_ref, pooled):
    b = pl.program_id(0)
    z = jnp.concatenate([z_ref[0], z_ref[1]], axis=-1)
    a = jnp.maximum(
        jnp.dot(z, wa_ref[...], preferred_element_type=jnp.float32) + ba_ref[...], 0.0)
    h = jnp.maximum(
        jnp.dot(a, wb_ref[...], preferred_element_type=jnp.float32) + bb_ref[...], 0.0)
    onehot_t = (lax.broadcasted_iota(jnp.int32, (_G, _BN), 0)
                == batch_ref[0]).astype(jnp.float32)
    contrib = jnp.dot(onehot_t, h, preferred_element_type=jnp.float32)

    @pl.when(b == 0)
    def _():
        pooled[...] = jnp.zeros_like(pooled)

    pooled[...] += contrib

    @pl.when(b == _NB - 1)
    def _():
        out_ref[...] = jnp.dot(
            pooled[...], wc_ref[...], preferred_element_type=jnp.float32) + bc_ref[...]


def _tc_final(z, batch_r, W2a, b2a, W2b, b2b, Wc, bc):
    return pl.pallas_call(
        _final_body,
        grid=(_NB,),
        in_specs=[
            pl.BlockSpec((_NC, _BN, _HALF), lambda b: (0, b, 0)),
            pl.BlockSpec((1, 1, _BN), lambda b: (b, 0, 0)),
            pl.BlockSpec((_D, _D), lambda b: (0, 0)),
            pl.BlockSpec((1, _D), lambda b: (0, 0)),
            pl.BlockSpec((_D, _D), lambda b: (0, 0)),
            pl.BlockSpec((1, _D), lambda b: (0, 0)),
            pl.BlockSpec((_D, _C), lambda b: (0, 0)),
            pl.BlockSpec((1, _C), lambda b: (0, 0)),
        ],
        out_specs=pl.BlockSpec((_G, _C), lambda b: (0, 0)),
        out_shape=jax.ShapeDtypeStruct((_G, _C), jnp.float32),
        scratch_shapes=[pltpu.VMEM((_G, _D), jnp.float32)],
    )(z, batch_r, W2a, b2a.reshape(1, _D), W2b, b2b.reshape(1, _D),
      Wc, bc.reshape(1, _C))


def kernel(x, edge_index, batch, W1a, b1a, W1b, b1b, W2a, b2a, W2b, b2b, Wc, bc):
    src = edge_index[0]
    dst = edge_index[1]
    src2 = jnp.concatenate([src, src + _N])
    x2 = jnp.concatenate([x[:, :_HALF], x[:, _HALF:]], axis=0)
    z1 = _sc_agg(x2, src2, dst)
    h1 = _tc_mlp(z1, W1a, b1a, W1b, b1b)
    z2 = _sc_agg(h1.reshape(_NC * _N, _HALF), src2, dst)
    batch_r = batch.reshape(_NB, 1, _BN)
    return _tc_final(z2, batch_r, W2a, b2a, W2b, b2b, Wc, bc)


# speedup vs baseline: 8.1161x; 2.5173x over previous
import functools

import jax
import jax.numpy as jnp
from jax import lax
from jax.experimental import pallas as pl
from jax.experimental.pallas import tpu as pltpu
from jax.experimental.pallas import tpu_sc as plsc

_N = 10000
_E = 320000
_D = 128
_HALF = 64
_G = 64
_C = 10
_NC = 2
_NS = 16
_K = 80
_EPT = _E // _NS
_CH = _EPT // _K
_RPT = 624
_TAIL0 = _NS * _RPT
_TAILN = _N - _TAIL0

_BN = 400
_NB = _N // _BN

_sc_mesh = plsc.VectorSubcoreMesh(core_axis_name="c", subcore_axis_name="s")


@functools.partial(
    pl.kernel,
    out_type=jax.ShapeDtypeStruct((_NC, _N, _HALF), jnp.float32),
    mesh=_sc_mesh,
    scratch_types=[
        pltpu.VMEM_SHARED((_N, _HALF), jnp.float32),
        pltpu.VMEM((_CH, _K), jnp.int32),
        pltpu.VMEM((_CH, _K), jnp.int32),
        pltpu.VMEM((_K, _HALF), jnp.float32),
        pltpu.VMEM((_K, _HALF), jnp.float32),
        pltpu.SemaphoreType.DMA,
        pltpu.SemaphoreType.DMA,
    ],
    compiler_params=pltpu.CompilerParams(use_tc_tiling_on_sc=False),
)
def _sc_agg(h2_hbm, src4_hbm, dst3_hbm, z_hbm, acc, sidx, didx,
            rows0, rows1, sem0, sem1):
    c = lax.axis_index("c")
    s = lax.axis_index("s")
    r0 = s * _RPT
    pltpu.sync_copy(src4_hbm.at[c, s], sidx)
    pltpu.sync_copy(dst3_hbm.at[s], didx)
    pltpu.sync_copy(h2_hbm.at[pl.ds(c * _N + r0, _RPT)], acc.at[pl.ds(r0, _RPT)])

    @pl.when(s == 0)
    def _():
        pltpu.sync_copy(h2_hbm.at[pl.ds(c * _N + _TAIL0, _TAILN)],
                        acc.at[pl.ds(_TAIL0, _TAILN)])

    plsc.subcore_barrier()

    pltpu.async_copy(h2_hbm.at[sidx.at[0]], rows0, sem0)

    def body(t, carry):
        j0 = 2 * t
        pltpu.async_copy(h2_hbm.at[sidx.at[j0 + 1]], rows1, sem1)
        pltpu.make_async_copy(h2_hbm.at[sidx.at[j0]], rows0, sem0).wait()
        pltpu.sync_copy(rows0, acc.at[didx.at[j0]], add=True)

        @pl.when(t < _CH // 2 - 1)
        def _():
            pltpu.async_copy(h2_hbm.at[sidx.at[j0 + 2]], rows0, sem0)

        pltpu.make_async_copy(h2_hbm.at[sidx.at[j0 + 1]], rows1, sem1).wait()
        pltpu.sync_copy(rows1, acc.at[didx.at[j0 + 1]], add=True)
        return carry

    lax.fori_loop(0, _CH // 2, body, 0)
    plsc.subcore_barrier()
    pltpu.sync_copy(acc.at[pl.ds(r0, _RPT)], z_hbm.at[c, pl.ds(r0, _RPT)])

    @pl.when(s == 0)
    def _():
        pltpu.sync_copy(acc.at[pl.ds(_TAIL0, _TAILN)],
                        z_hbm.at[c, pl.ds(_TAIL0, _TAILN)])


def _mlp_body(z_ref, wa_ref, ba_ref, wb_ref, bb_ref, h_ref):
    z = jnp.concatenate([z_ref[0], z_ref[1]], axis=-1)
    a = jnp.maximum(
        jnp.dot(z, wa_ref[...], preferred_element_type=jnp.float32) + ba_ref[...], 0.0)
    h = jnp.maximum(
        jnp.dot(a, wb_ref[...], preferred_element_type=jnp.float32) + bb_ref[...], 0.0)
    h_ref[0] = h[:, :_HALF]
    h_ref[1] = h[:, _HALF:]


def _tc_mlp(z, Wa, ba, Wb, bb):
    return pl.pallas_call(
        _mlp_body,
        grid=(_NB,),
        in_specs=[
            pl.BlockSpec((_NC, _BN, _HALF), lambda b: (0, b, 0)),
            pl.BlockSpec((_D, _D), lambda b: (0, 0)),
            pl.BlockSpec((1, _D), lambda b: (0, 0)),
            pl.BlockSpec((_D, _D), lambda b: (0, 0)),
            pl.BlockSpec((1, _D), lambda b: (0, 0)),
        ],
        out_specs=pl.BlockSpec((_NC, _BN, _HALF), lambda b: (0, b, 0)),
        out_shape=jax.ShapeDtypeStruct((_NC, _N, _HALF), jnp.float32),
    )(z, Wa, ba.reshape(1, _D), Wb, bb.reshape(1, _D))


def _final_body(z_ref, batch_ref, wa_ref, ba_ref, wb_ref, bb_ref, wc_ref,
                bc_ref, out_ref, pooled):
    b = pl.program_id(0)
    z = jnp.concatenate([z_ref[0], z_ref[1]], axis=-1)
    a = jnp.maximum(
        jnp.dot(z, wa_ref[...], preferred_element_type=jnp.float32) + ba_ref[...], 0.0)
    h = jnp.maximum(
        jnp.dot(a, wb_ref[...], preferred_element_type=jnp.float32) + bb_ref[...], 0.0)
    onehot_t = (lax.broadcasted_iota(jnp.int32, (_G, _BN), 0)
                == batch_ref[0]).astype(jnp.float32)
    contrib = jnp.dot(onehot_t, h, preferred_element_type=jnp.float32)

    @pl.when(b == 0)
    def _():
        pooled[...] = jnp.zeros_like(pooled)

    pooled[...] += contrib

    @pl.when(b == _NB - 1)
    def _():
        out_ref[...] = jnp.dot(
            pooled[...], wc_ref[...], preferred_element_type=jnp.float32) + bc_ref[...]


def _tc_final(z, batch_r, W2a, b2a, W2b, b2b, Wc, bc):
    return pl.pallas_call(
        _final_body,
        grid=(_NB,),
        in_specs=[
            pl.BlockSpec((_NC, _BN, _HALF), lambda b: (0, b, 0)),
            pl.BlockSpec((1, 1, _BN), lambda b: (b, 0, 0)),
            pl.BlockSpec((_D, _D), lambda b: (0, 0)),
            pl.BlockSpec((1, _D), lambda b: (0, 0)),
            pl.BlockSpec((_D, _D), lambda b: (0, 0)),
            pl.BlockSpec((1, _D), lambda b: (0, 0)),
            pl.BlockSpec((_D, _C), lambda b: (0, 0)),
            pl.BlockSpec((1, _C), lambda b: (0, 0)),
        ],
        out_specs=pl.BlockSpec((_G, _C), lambda b: (0, 0)),
        out_shape=jax.ShapeDtypeStruct((_G, _C), jnp.float32),
        scratch_shapes=[pltpu.VMEM((_G, _D), jnp.float32)],
    )(z, batch_r, W2a, b2a.reshape(1, _D), W2b, b2b.reshape(1, _D),
      Wc, bc.reshape(1, _C))


def kernel(x, edge_index, batch, W1a, b1a, W1b, b1b, W2a, b2a, W2b, b2b, Wc, bc):
    src = edge_index[0]
    dst = edge_index[1]
    src4 = jnp.concatenate([src, src + _N]).reshape(_NC, _NS, _CH, _K)
    dst3 = dst.reshape(_NS, _CH, _K)
    x2 = jnp.concatenate([x[:, :_HALF], x[:, _HALF:]], axis=0)
    z1 = _sc_agg(x2, src4, dst3)
    h1 = _tc_mlp(z1, W1a, b1a, W1b, b1b)
    z2 = _sc_agg(h1.reshape(_NC * _N, _HALF), src4, dst3)
    batch_r = batch.reshape(_NB, 1, _BN)
    return _tc_final(z2, batch_r, W2a, b2a, W2b, b2b, Wc, bc)


# speedup vs baseline: 10.7755x; 1.3277x over previous
import functools

import jax
import jax.numpy as jnp
from jax import lax
from jax.experimental import pallas as pl
from jax.experimental.pallas import tpu as pltpu
from jax.experimental.pallas import tpu_sc as plsc

_N = 10000
_E = 320000
_D = 128
_HALF = 64
_G = 64
_C = 10
_NC = 2
_NS = 16
_K = 80
_EPT = _E // _NS
_CH = _EPT // _K
_RPT = 624
_TAIL0 = _NS * _RPT
_TAILN = _N - _TAIL0

_NBUF = 8
_SLACK = 5

_BN = 400
_NB = _N // _BN

_sc_mesh = plsc.VectorSubcoreMesh(core_axis_name="c", subcore_axis_name="s")


@functools.partial(
    pl.kernel,
    out_type=jax.ShapeDtypeStruct((_NC, _N, _HALF), jnp.float32),
    mesh=_sc_mesh,
    scratch_types=[
        pltpu.VMEM_SHARED((_N, _HALF), jnp.float32),
        pltpu.VMEM((_CH, _K), jnp.int32),
        pltpu.VMEM((_CH, _K), jnp.int32),
    ]
    + [pltpu.VMEM((_K, _HALF), jnp.float32)] * _NBUF
    + [pltpu.SemaphoreType.DMA] * (2 * _NBUF),
    compiler_params=pltpu.CompilerParams(use_tc_tiling_on_sc=False),
)
def _sc_agg(h2_hbm, src4_hbm, dst3_hbm, z_hbm, acc, sidx, didx, *rest):
    rows = rest[0:_NBUF]
    gsem = rest[_NBUF:2 * _NBUF]
    ssem = rest[2 * _NBUF:3 * _NBUF]
    c = lax.axis_index("c")
    s = lax.axis_index("s")
    r0 = s * _RPT
    pltpu.sync_copy(src4_hbm.at[c, s], sidx)
    pltpu.sync_copy(dst3_hbm.at[s], didx)
    pltpu.sync_copy(h2_hbm.at[pl.ds(c * _N + r0, _RPT)], acc.at[pl.ds(r0, _RPT)])

    @pl.when(s == 0)
    def _():
        pltpu.sync_copy(h2_hbm.at[pl.ds(c * _N + _TAIL0, _TAILN)],
                        acc.at[pl.ds(_TAIL0, _TAILN)])

    plsc.subcore_barrier()

    def g_issue(j, b):
        pltpu.async_copy(h2_hbm.at[sidx.at[j]], rows[b], gsem[b])

    def g_wait(j, b):
        pltpu.make_async_copy(h2_hbm.at[sidx.at[j]], rows[b], gsem[b]).wait()

    def s_issue(j, b):
        pltpu.async_copy(rows[b], acc.at[didx.at[j]], ssem[b], add=True)

    def s_wait(j, b):
        pltpu.make_async_copy(rows[b], acc.at[didx.at[j]], ssem[b]).wait()

    for b in range(_SLACK):
        g_issue(b, b)
    for j in range(_SLACK):
        g_wait(j, j % _NBUF)
        s_issue(j, j % _NBUF)
        b2 = (j + _SLACK) % _NBUF
        if j + _SLACK >= _NBUF:
            s_wait(j + _SLACK - _NBUF, b2)
        g_issue(j + _SLACK, b2)

    def outer(t, carry):
        j0 = _SLACK + t * _NBUF
        for bp in range(_NBUF):
            j = j0 + bp
            b = (_SLACK + bp) % _NBUF
            g_wait(j, b)
            s_issue(j, b)
            b2 = (b + _SLACK) % _NBUF
            s_wait(j + _SLACK - _NBUF, b2)
            g_issue(j + _SLACK, b2)
        return carry

    lax.fori_loop(0, (_CH - 2 * _SLACK) // _NBUF, outer, 0)

    for k in range(_SLACK):
        j = _CH - _SLACK + k
        g_wait(j, j % _NBUF)
        s_issue(j, j % _NBUF)
    for b in range(_NBUF):
        s_wait(_CH - _NBUF + b, b)
    plsc.subcore_barrier()
    pltpu.sync_copy(acc.at[pl.ds(r0, _RPT)], z_hbm.at[c, pl.ds(r0, _RPT)])

    @pl.when(s == 0)
    def _():
        pltpu.sync_copy(acc.at[pl.ds(_TAIL0, _TAILN)],
                        z_hbm.at[c, pl.ds(_TAIL0, _TAILN)])


def _mlp_body(z_ref, wa_ref, ba_ref, wb_ref, bb_ref, h_ref):
    z = jnp.concatenate([z_ref[0], z_ref[1]], axis=-1)
    a = jnp.maximum(
        jnp.dot(z, wa_ref[...], preferred_element_type=jnp.float32) + ba_ref[...], 0.0)
    h = jnp.maximum(
        jnp.dot(a, wb_ref[...], preferred_element_type=jnp.float32) + bb_ref[...], 0.0)
    h_ref[0] = h[:, :_HALF]
    h_ref[1] = h[:, _HALF:]


def _tc_mlp(z, Wa, ba, Wb, bb):
    return pl.pallas_call(
        _mlp_body,
        grid=(_NB,),
        in_specs=[
            pl.BlockSpec((_NC, _BN, _HALF), lambda b: (0, b, 0)),
            pl.BlockSpec((_D, _D), lambda b: (0, 0)),
            pl.BlockSpec((1, _D), lambda b: (0, 0)),
            pl.BlockSpec((_D, _D), lambda b: (0, 0)),
            pl.BlockSpec((1, _D), lambda b: (0, 0)),
        ],
        out_specs=pl.BlockSpec((_NC, _BN, _HALF), lambda b: (0, b, 0)),
        out_shape=jax.ShapeDtypeStruct((_NC, _N, _HALF), jnp.float32),
    )(z, Wa, ba.reshape(1, _D), Wb, bb.reshape(1, _D))


def _final_body(z_ref, batch_ref, wa_ref, ba_ref, wb_ref, bb_ref, wc_ref,
                bc_ref, out_ref, pooled):
    b = pl.program_id(0)
    z = jnp.concatenate([z_ref[0], z_ref[1]], axis=-1)
    a = jnp.maximum(
        jnp.dot(z, wa_ref[...], preferred_element_type=jnp.float32) + ba_ref[...], 0.0)
    h = jnp.maximum(
        jnp.dot(a, wb_ref[...], preferred_element_type=jnp.float32) + bb_ref[...], 0.0)
    onehot_t = (lax.broadcasted_iota(jnp.int32, (_G, _BN), 0)
                == batch_ref[0]).astype(jnp.float32)
    contrib = jnp.dot(onehot_t, h, preferred_element_type=jnp.float32)

    @pl.when(b == 0)
    def _():
        pooled[...] = jnp.zeros_like(pooled)

    pooled[...] += contrib

    @pl.when(b == _NB - 1)
    def _():
        out_ref[...] = jnp.dot(
            pooled[...], wc_ref[...], preferred_element_type=jnp.float32) + bc_ref[...]


def _tc_final(z, batch_r, W2a, b2a, W2b, b2b, Wc, bc):
    return pl.pallas_call(
        _final_body,
        grid=(_NB,),
        in_specs=[
            pl.BlockSpec((_NC, _BN, _HALF), lambda b: (0, b, 0)),
            pl.BlockSpec((1, 1, _BN), lambda b: (b, 0, 0)),
            pl.BlockSpec((_D, _D), lambda b: (0, 0)),
            pl.BlockSpec((1, _D), lambda b: (0, 0)),
            pl.BlockSpec((_D, _D), lambda b: (0, 0)),
            pl.BlockSpec((1, _D), lambda b: (0, 0)),
            pl.BlockSpec((_D, _C), lambda b: (0, 0)),
            pl.BlockSpec((1, _C), lambda b: (0, 0)),
        ],
        out_specs=pl.BlockSpec((_G, _C), lambda b: (0, 0)),
        out_shape=jax.ShapeDtypeStruct((_G, _C), jnp.float32),
        scratch_shapes=[pltpu.VMEM((_G, _D), jnp.float32)],
    )(z, batch_r, W2a, b2a.reshape(1, _D), W2b, b2b.reshape(1, _D),
      Wc, bc.reshape(1, _C))


def kernel(x, edge_index, batch, W1a, b1a, W1b, b1b, W2a, b2a, W2b, b2b, Wc, bc):
    src = edge_index[0]
    dst = edge_index[1]
    src4 = jnp.concatenate([src, src + _N]).reshape(_NC, _NS, _CH, _K)
    dst3 = dst.reshape(_NS, _CH, _K)
    x2 = jnp.concatenate([x[:, :_HALF], x[:, _HALF:]], axis=0)
    z1 = _sc_agg(x2, src4, dst3)
    h1 = _tc_mlp(z1, W1a, b1a, W1b, b1b)
    z2 = _sc_agg(h1.reshape(_NC * _N, _HALF), src4, dst3)
    batch_r = batch.reshape(_NB, 1, _BN)
    return _tc_final(z2, batch_r, W2a, b2a, W2b, b2b, Wc, bc)


# speedup vs baseline: 13.4731x; 1.2504x over previous
import functools

import jax
import jax.numpy as jnp
from jax import lax
from jax.experimental import pallas as pl
from jax.experimental.pallas import tpu as pltpu
from jax.experimental.pallas import tpu_sc as plsc

_N = 10000
_E = 320000
_D = 128
_G = 64
_C = 10
_NC = 2
_NS = 16
_K = 40
_EPT = _E // _NC // _NS
_CH = _EPT // _K
_RPT = 624
_TAIL0 = _NS * _RPT
_TAILN = _N - _TAIL0

_NBUF = 6
_SLACK = 5

_BN = 400
_NB = _N // _BN

_sc_mesh = plsc.VectorSubcoreMesh(core_axis_name="c", subcore_axis_name="s")


@functools.partial(
    pl.kernel,
    out_type=jax.ShapeDtypeStruct((_NC, _N, _D), jnp.float32),
    mesh=_sc_mesh,
    scratch_types=[
        pltpu.VMEM_SHARED((_N, _D), jnp.float32),
        pltpu.VMEM((_CH, _K), jnp.int32),
        pltpu.VMEM((_CH, _K), jnp.int32),
    ]
    + [pltpu.VMEM((_K, _D), jnp.float32)] * _NBUF
    + [pltpu.SemaphoreType.DMA] * (2 * _NBUF),
    compiler_params=pltpu.CompilerParams(use_tc_tiling_on_sc=False),
)
def _sc_agg(h_hbm, src3_hbm, dst3_hbm, p_hbm, acc, sidx, didx, *rest):
    rows = rest[0:_NBUF]
    gsem = rest[_NBUF:2 * _NBUF]
    ssem = rest[2 * _NBUF:3 * _NBUF]
    c = lax.axis_index("c")
    s = lax.axis_index("s")
    r0 = s * _RPT
    pltpu.sync_copy(src3_hbm.at[c, s], sidx)
    pltpu.sync_copy(dst3_hbm.at[c, s], didx)
    pltpu.sync_copy(h_hbm.at[pl.ds(r0, _RPT)], acc.at[pl.ds(r0, _RPT)])

    @pl.when(s == 0)
    def _():
        pltpu.sync_copy(h_hbm.at[pl.ds(_TAIL0, _TAILN)],
                        acc.at[pl.ds(_TAIL0, _TAILN)])

    plsc.subcore_barrier()

    def g_issue(j, b):
        pltpu.async_copy(h_hbm.at[sidx.at[j]], rows[b], gsem[b])

    def g_wait(j, b):
        pltpu.make_async_copy(h_hbm.at[sidx.at[j]], rows[b], gsem[b]).wait()

    def s_issue(j, b):
        pltpu.async_copy(rows[b], acc.at[didx.at[j]], ssem[b], add=True)

    def s_wait(j, b):
        pltpu.make_async_copy(rows[b], acc.at[didx.at[j]], ssem[b]).wait()

    for b in range(_SLACK):
        g_issue(b, b)
    for j in range(_SLACK):
        g_wait(j, j % _NBUF)
        s_issue(j, j % _NBUF)
        b2 = (j + _SLACK) % _NBUF
        if j + _SLACK >= _NBUF:
            s_wait(j + _SLACK - _NBUF, b2)
        g_issue(j + _SLACK, b2)

    def outer(t, carry):
        j0 = _SLACK + t * _NBUF
        for bp in range(_NBUF):
            j = j0 + bp
            b = (_SLACK + bp) % _NBUF
            g_wait(j, b)
            s_issue(j, b)
            b2 = (b + _SLACK) % _NBUF
            s_wait(j + _SLACK - _NBUF, b2)
            g_issue(j + _SLACK, b2)
        return carry

    lax.fori_loop(0, (_CH - 2 * _SLACK) // _NBUF, outer, 0)

    for k in range(_SLACK):
        j = _CH - _SLACK + k
        g_wait(j, j % _NBUF)
        s_issue(j, j % _NBUF)
    for b in range(_NBUF):
        s_wait(_CH - _NBUF + b, b)
    plsc.subcore_barrier()
    pltpu.sync_copy(acc.at[pl.ds(r0, _RPT)], p_hbm.at[c, pl.ds(r0, _RPT)])

    @pl.when(s == 0)
    def _():
        pltpu.sync_copy(acc.at[pl.ds(_TAIL0, _TAILN)],
                        p_hbm.at[c, pl.ds(_TAIL0, _TAILN)])


def _mlp_body(p0_ref, p1_ref, h_ref, wa_ref, ba_ref, wb_ref, bb_ref, o_ref):
    z = p0_ref[0] + p1_ref[0] - h_ref[...]
    a = jnp.maximum(
        jnp.dot(z, wa_ref[...], preferred_element_type=jnp.float32) + ba_ref[...], 0.0)
    o_ref[...] = jnp.maximum(
        jnp.dot(a, wb_ref[...], preferred_element_type=jnp.float32) + bb_ref[...], 0.0)


def _tc_mlp(p, h, Wa, ba, Wb, bb):
    return pl.pallas_call(
        _mlp_body,
        grid=(_NB,),
        in_specs=[
            pl.BlockSpec((1, _BN, _D), lambda b: (0, b, 0)),
            pl.BlockSpec((1, _BN, _D), lambda b: (1, b, 0)),
            pl.BlockSpec((_BN, _D), lambda b: (b, 0)),
            pl.BlockSpec((_D, _D), lambda b: (0, 0)),
            pl.BlockSpec((1, _D), lambda b: (0, 0)),
            pl.BlockSpec((_D, _D), lambda b: (0, 0)),
            pl.BlockSpec((1, _D), lambda b: (0, 0)),
        ],
        out_specs=pl.BlockSpec((_BN, _D), lambda b: (b, 0)),
        out_shape=jax.ShapeDtypeStruct((_N, _D), jnp.float32),
    )(p, p, h, Wa, ba.reshape(1, _D), Wb, bb.reshape(1, _D))


def _final_body(p0_ref, p1_ref, h_ref, batch_ref, wa_ref, ba_ref, wb_ref,
                bb_ref, wc_ref, bc_ref, out_ref, pooled):
    b = pl.program_id(0)
    z = p0_ref[0] + p1_ref[0] - h_ref[...]
    a = jnp.maximum(
        jnp.dot(z, wa_ref[...], preferred_element_type=jnp.float32) + ba_ref[...], 0.0)
    h = jnp.maximum(
        jnp.dot(a, wb_ref[...], preferred_element_type=jnp.float32) + bb_ref[...], 0.0)
    onehot_t = (lax.broadcasted_iota(jnp.int32, (_G, _BN), 0)
                == batch_ref[0]).astype(jnp.float32)
    contrib = jnp.dot(onehot_t, h, preferred_element_type=jnp.float32)

    @pl.when(b == 0)
    def _():
        pooled[...] = jnp.zeros_like(pooled)

    pooled[...] += contrib

    @pl.when(b == _NB - 1)
    def _():
        out_ref[...] = jnp.dot(
            pooled[...], wc_ref[...], preferred_element_type=jnp.float32) + bc_ref[...]


def _tc_final(p, h, batch_r, W2a, b2a, W2b, b2b, Wc, bc):
    return pl.pallas_call(
        _final_body,
        grid=(_NB,),
        in_specs=[
            pl.BlockSpec((1, _BN, _D), lambda b: (0, b, 0)),
            pl.BlockSpec((1, _BN, _D), lambda b: (1, b, 0)),
            pl.BlockSpec((_BN, _D), lambda b: (b, 0)),
            pl.BlockSpec((1, 1, _BN), lambda b: (b, 0, 0)),
            pl.BlockSpec((_D, _D), lambda b: (0, 0)),
            pl.BlockSpec((1, _D), lambda b: (0, 0)),
            pl.BlockSpec((_D, _D), lambda b: (0, 0)),
            pl.BlockSpec((1, _D), lambda b: (0, 0)),
            pl.BlockSpec((_D, _C), lambda b: (0, 0)),
            pl.BlockSpec((1, _C), lambda b: (0, 0)),
        ],
        out_specs=pl.BlockSpec((_G, _C), lambda b: (0, 0)),
        out_shape=jax.ShapeDtypeStruct((_G, _C), jnp.float32),
        scratch_shapes=[pltpu.VMEM((_G, _D), jnp.float32)],
    )(p, p, h, batch_r, W2a, b2a.reshape(1, _D), W2b, b2b.reshape(1, _D),
      Wc, bc.reshape(1, _C))


def kernel(x, edge_index, batch, W1a, b1a, W1b, b1b, W2a, b2a, W2b, b2b, Wc, bc):
    src3 = edge_index[0].reshape(_NC, _NS, _CH, _K)
    dst3 = edge_index[1].reshape(_NC, _NS, _CH, _K)
    p1 = _sc_agg(x, src3, dst3)
    h1 = _tc_mlp(p1, x, W1a, b1a, W1b, b1b)
    p2 = _sc_agg(h1, src3, dst3)
    batch_r = batch.reshape(_NB, 1, _BN)
    return _tc_final(p2, h1, batch_r, W2a, b2a, W2b, b2b, Wc, bc)


# speedup vs baseline: 15.4446x; 1.1463x over previous
import functools

import jax
import jax.numpy as jnp
from jax import lax
from jax.experimental import pallas as pl
from jax.experimental.pallas import tpu as pltpu
from jax.experimental.pallas import tpu_sc as plsc

_N = 10000
_E = 320000
_D = 128
_G = 64
_C = 10
_NC = 2
_NS = 16
_K = 40
_EPT = _E // _NC // _NS
_CH = _EPT // _K
_RPT = 624
_TAIL0 = _NS * _RPT
_TAILN = _N - _TAIL0

_NBUF = 6
_SLACK = 5

_BN = 2000
_NB = _N // _BN

_sc_mesh = plsc.VectorSubcoreMesh(core_axis_name="c", subcore_axis_name="s")


@functools.partial(
    pl.kernel,
    out_type=jax.ShapeDtypeStruct((_NC, _N, _D), jnp.float32),
    mesh=_sc_mesh,
    scratch_types=[
        pltpu.VMEM_SHARED((_N, _D), jnp.float32),
        pltpu.VMEM((_CH, _K), jnp.int32),
        pltpu.VMEM((_CH, _K), jnp.int32),
    ]
    + [pltpu.VMEM((_K, _D), jnp.float32)] * _NBUF
    + [pltpu.SemaphoreType.DMA] * (2 * _NBUF),
    compiler_params=pltpu.CompilerParams(use_tc_tiling_on_sc=False),
)
def _sc_agg(h_hbm, ei5_hbm, p_hbm, acc, sidx, didx, *rest):
    rows = rest[0:_NBUF]
    gsem = rest[_NBUF:2 * _NBUF]
    ssem = rest[2 * _NBUF:3 * _NBUF]
    c = lax.axis_index("c")
    s = lax.axis_index("s")
    r0 = s * _RPT
    pltpu.sync_copy(ei5_hbm.at[0, c, s], sidx)

    def g_issue(j, b):
        pltpu.async_copy(h_hbm.at[sidx.at[j]], rows[b], gsem[b])

    def g_wait(j, b):
        pltpu.make_async_copy(h_hbm.at[sidx.at[j]], rows[b], gsem[b]).wait()

    def s_issue(j, b):
        pltpu.async_copy(rows[b], acc.at[didx.at[j]], ssem[b], add=True)

    def s_wait(j, b):
        pltpu.make_async_copy(rows[b], acc.at[didx.at[j]], ssem[b]).wait()

    for b in range(_SLACK):
        g_issue(b, b)

    pltpu.sync_copy(ei5_hbm.at[1, c, s], didx)
    pltpu.sync_copy(h_hbm.at[pl.ds(r0, _RPT)], acc.at[pl.ds(r0, _RPT)])

    @pl.when(s == 0)
    def _():
        pltpu.sync_copy(h_hbm.at[pl.ds(_TAIL0, _TAILN)],
                        acc.at[pl.ds(_TAIL0, _TAILN)])

    plsc.subcore_barrier()

    for j in range(_SLACK):
        g_wait(j, j % _NBUF)
        s_issue(j, j % _NBUF)
        b2 = (j + _SLACK) % _NBUF
        if j + _SLACK >= _NBUF:
            s_wait(j + _SLACK - _NBUF, b2)
        g_issue(j + _SLACK, b2)

    def outer(t, carry):
        j0 = _SLACK + t * _NBUF
        for bp in range(_NBUF):
            j = j0 + bp
            b = (_SLACK + bp) % _NBUF
            g_wait(j, b)
            s_issue(j, b)
            b2 = (b + _SLACK) % _NBUF
            s_wait(j + _SLACK - _NBUF, b2)
            g_issue(j + _SLACK, b2)
        return carry

    lax.fori_loop(0, (_CH - 2 * _SLACK) // _NBUF, outer, 0)

    for k in range(_SLACK):
        j = _CH - _SLACK + k
        g_wait(j, j % _NBUF)
        s_issue(j, j % _NBUF)
    for b in range(_NBUF):
        s_wait(_CH - _NBUF + b, b)
    plsc.subcore_barrier()
    pltpu.sync_copy(acc.at[pl.ds(r0, _RPT)], p_hbm.at[c, pl.ds(r0, _RPT)])

    @pl.when(s == 0)
    def _():
        pltpu.sync_copy(acc.at[pl.ds(_TAIL0, _TAILN)],
                        p_hbm.at[c, pl.ds(_TAIL0, _TAILN)])


def _mlp_body(p0_ref, p1_ref, h_ref, wa_ref, ba_ref, wb_ref, bb_ref, o_ref):
    z = p0_ref[0] + p1_ref[0] - h_ref[...]
    a = jnp.maximum(
        jnp.dot(z, wa_ref[...], preferred_element_type=jnp.float32) + ba_ref[...], 0.0)
    o_ref[...] = jnp.maximum(
        jnp.dot(a, wb_ref[...], preferred_element_type=jnp.float32) + bb_ref[...], 0.0)


def _tc_mlp(p, h, Wa, ba, Wb, bb):
    return pl.pallas_call(
        _mlp_body,
        grid=(_NB,),
        in_specs=[
            pl.BlockSpec((1, _BN, _D), lambda b: (0, b, 0)),
            pl.BlockSpec((1, _BN, _D), lambda b: (1, b, 0)),
            pl.BlockSpec((_BN, _D), lambda b: (b, 0)),
            pl.BlockSpec((_D, _D), lambda b: (0, 0)),
            pl.BlockSpec((1, _D), lambda b: (0, 0)),
            pl.BlockSpec((_D, _D), lambda b: (0, 0)),
            pl.BlockSpec((1, _D), lambda b: (0, 0)),
        ],
        out_specs=pl.BlockSpec((_BN, _D), lambda b: (b, 0)),
        out_shape=jax.ShapeDtypeStruct((_N, _D), jnp.float32),
    )(p, p, h, Wa, ba.reshape(1, _D), Wb, bb.reshape(1, _D))


def _final_body(p0_ref, p1_ref, h_ref, batch_ref, wa_ref, ba_ref, wb_ref,
                bb_ref, wc_ref, bc_ref, out_ref, pooled):
    b = pl.program_id(0)
    z = p0_ref[0] + p1_ref[0] - h_ref[...]
    a = jnp.maximum(
        jnp.dot(z, wa_ref[...], preferred_element_type=jnp.float32) + ba_ref[...], 0.0)
    h = jnp.maximum(
        jnp.dot(a, wb_ref[...], preferred_element_type=jnp.float32) + bb_ref[...], 0.0)
    onehot_t = (lax.broadcasted_iota(jnp.int32, (_G, _BN), 0)
                == batch_ref[0]).astype(jnp.float32)
    contrib = jnp.dot(onehot_t, h, preferred_element_type=jnp.float32)

    @pl.when(b == 0)
    def _():
        pooled[...] = jnp.zeros_like(pooled)

    pooled[...] += contrib

    @pl.when(b == _NB - 1)
    def _():
        out_ref[...] = jnp.dot(
            pooled[...], wc_ref[...], preferred_element_type=jnp.float32) + bc_ref[...]


def _tc_final(p, h, batch_r, W2a, b2a, W2b, b2b, Wc, bc):
    return pl.pallas_call(
        _final_body,
        grid=(_NB,),
        in_specs=[
            pl.BlockSpec((1, _BN, _D), lambda b: (0, b, 0)),
            pl.BlockSpec((1, _BN, _D), lambda b: (1, b, 0)),
            pl.BlockSpec((_BN, _D), lambda b: (b, 0)),
            pl.BlockSpec((1, 1, _BN), lambda b: (b, 0, 0)),
            pl.BlockSpec((_D, _D), lambda b: (0, 0)),
            pl.BlockSpec((1, _D), lambda b: (0, 0)),
            pl.BlockSpec((_D, _D), lambda b: (0, 0)),
            pl.BlockSpec((1, _D), lambda b: (0, 0)),
            pl.BlockSpec((_D, _C), lambda b: (0, 0)),
            pl.BlockSpec((1, _C), lambda b: (0, 0)),
        ],
        out_specs=pl.BlockSpec((_G, _C), lambda b: (0, 0)),
        out_shape=jax.ShapeDtypeStruct((_G, _C), jnp.float32),
        scratch_shapes=[pltpu.VMEM((_G, _D), jnp.float32)],
    )(p, p, h, batch_r, W2a, b2a.reshape(1, _D), W2b, b2b.reshape(1, _D),
      Wc, bc.reshape(1, _C))


def kernel(x, edge_index, batch, W1a, b1a, W1b, b1b, W2a, b2a, W2b, b2b, Wc, bc):
    ei5 = edge_index.reshape(2, _NC, _NS, _CH, _K)
    p1 = _sc_agg(x, ei5)
    h1 = _tc_mlp(p1, x, W1a, b1a, W1b, b1b)
    p2 = _sc_agg(h1, ei5)
    batch_r = batch.reshape(_NB, 1, _BN)
    return _tc_final(p2, h1, batch_r, W2a, b2a, W2b, b2b, Wc, bc)
